# baseline (speedup 1.0000x reference)
import functools
import jax
import jax.numpy as jnp
from jax import lax
from jax.experimental import pallas as pl
from jax.experimental.pallas import tpu as pltpu
from jax.experimental.pallas import tpu_sc as plsc

_TBL = 3969
_TBLP = 4096
_OFF = 1984


def _rpe_sc_kernel(wt_hbm, out_hbm, col_v, w_v, buf_v, sem0, sem1, sem2):
    core = lax.axis_index("c")
    sub = lax.axis_index("s")
    h = sub
    half = core
    sems = (sem0, sem1, sem2)

    pltpu.sync_copy(wt_hbm.at[h], col_v)

    def revchunk(t, carry):
        w_v[pl.ds(16 * t, 16)] = lax.rev(col_v[pl.ds(3953 - 16 * t, 16)], (0,))
        return carry

    lax.fori_loop(0, 248, revchunk, 0)
    w_v[pl.ds(3953, 16)] = lax.rev(col_v[pl.ds(0, 16)], (0,))

    def fill(g, b):
        ri = 16 * half + g
        base_ri = _OFF - 63 * ri

        @plsc.parallel_loop(0, 32, 1, unroll=2)
        def row(ci):
            base_i = base_ri - ci
            for c in range(64):
                off = 63 * (c >> 1) + 16 * (c & 1)
                buf_v[b, ci, pl.ds(16 * c, 16)] = w_v[pl.ds(base_i + off, 16)]

        return ri

    for b in range(3):
        ri = fill(b, b)
        pltpu.async_copy(buf_v.at[b], out_hbm.at[h, pl.ds(ri * 32, 32)], sems[b])

    def body(g2, carry):
        g = 3 * g2
        for b in range(3):
            pltpu.make_async_copy(
                buf_v.at[b], out_hbm.at[h, pl.ds(0, 32)], sems[b]
            ).wait()
            ri = fill(g + b, b)
            pltpu.async_copy(buf_v.at[b], out_hbm.at[h, pl.ds(ri * 32, 32)], sems[b])
        return carry

    lax.fori_loop(1, 5, body, 0)

    pltpu.make_async_copy(buf_v.at[0], out_hbm.at[h, pl.ds(0, 32)], sems[0]).wait()
    ri = fill(15, 0)
    pltpu.async_copy(buf_v.at[0], out_hbm.at[h, pl.ds(ri * 32, 32)], sems[0])

    for b in range(3):
        pltpu.make_async_copy(
            buf_v.at[b], out_hbm.at[h, pl.ds(0, 32)], sems[b]
        ).wait()


@jax.jit
def kernel(rpe_bias):
    wt = jnp.pad(rpe_bias.T, ((0, 0), (0, _TBLP - _TBL)))
    mesh = plsc.VectorSubcoreMesh(core_axis_name="c", subcore_axis_name="s")
    run = functools.partial(
        pl.kernel,
        mesh=mesh,
        out_type=jax.ShapeDtypeStruct((16, 1024, 1024), jnp.float32),
        scratch_types=[
            pltpu.VMEM((_TBLP,), jnp.float32),
            pltpu.VMEM((_TBLP,), jnp.float32),
            pltpu.VMEM((3, 32, 1024), jnp.float32),
            pltpu.SemaphoreType.DMA,
            pltpu.SemaphoreType.DMA,
            pltpu.SemaphoreType.DMA,
        ],
    )(_rpe_sc_kernel)
    return run(wt)


# speedup vs baseline: 1.3099x; 1.3099x over previous
import functools
import jax
import jax.numpy as jnp
from jax import lax
from jax.experimental import pallas as pl
from jax.experimental.pallas import tpu as pltpu
from jax.experimental.pallas import tpu_sc as plsc

_TBL = 3969
_TBLP = 4096
_OFF = 1984


def _rpe_sc_kernel(wt_hbm, out_hbm, col_v, w_v, buf_v, sem0, sem1, sem2):
    core = lax.axis_index("c")
    sub = lax.axis_index("s")
    h = sub
    half = core
    sems = (sem0, sem1, sem2)

    pltpu.sync_copy(wt_hbm.at[h], col_v)

    def revchunk(t, carry):
        w_v[pl.ds(16 * t, 16)] = lax.rev(col_v[pl.ds(3953 - 16 * t, 16)], (0,))
        return carry

    lax.fori_loop(0, 248, revchunk, 0)
    w_v[pl.ds(3953, 16)] = lax.rev(col_v[pl.ds(0, 16)], (0,))

    def fill(g, b):
        ri = 16 * half + g
        base_ri = _OFF - 63 * ri

        @plsc.parallel_loop(0, 256, 1, unroll=2)
        def piece(t):
            ci = t >> 3
            c8 = t & 7
            base_t = base_ri - ci + 252 * c8
            for cc in range(8):
                off = 63 * (cc >> 1) + 16 * (cc & 1)
                buf_v[b, ci, pl.ds(128 * c8 + 16 * cc, 16)] = w_v[
                    pl.ds(base_t + off, 16)
                ]

        return ri

    for b in range(2):
        ri = fill(b, b)
        pltpu.async_copy(buf_v.at[b], out_hbm.at[h, pl.ds(ri * 32, 32)], sems[b])

    def body(g2, carry):
        g = 2 * g2
        for b in range(2):
            pltpu.make_async_copy(
                buf_v.at[b], out_hbm.at[h, pl.ds(0, 32)], sems[b]
            ).wait()
            ri = fill(g + b, b)
            pltpu.async_copy(buf_v.at[b], out_hbm.at[h, pl.ds(ri * 32, 32)], sems[b])
        return carry

    lax.fori_loop(1, 8, body, 0)

    for b in range(2):
        pltpu.make_async_copy(
            buf_v.at[b], out_hbm.at[h, pl.ds(0, 32)], sems[b]
        ).wait()


@jax.jit
def kernel(rpe_bias):
    wt = jnp.pad(rpe_bias.T, ((0, 0), (0, _TBLP - _TBL)))
    mesh = plsc.VectorSubcoreMesh(core_axis_name="c", subcore_axis_name="s")
    run = functools.partial(
        pl.kernel,
        mesh=mesh,
        out_type=jax.ShapeDtypeStruct((16, 1024, 1024), jnp.float32),
        scratch_types=[
            pltpu.VMEM((_TBLP,), jnp.float32),
            pltpu.VMEM((_TBLP,), jnp.float32),
            pltpu.VMEM((3, 32, 1024), jnp.float32),
            pltpu.SemaphoreType.DMA,
            pltpu.SemaphoreType.DMA,
            pltpu.SemaphoreType.DMA,
        ],
    )(_rpe_sc_kernel)
    return run(wt)
